# Initial kernel scaffold; baseline (speedup 1.0000x reference)
import jax
import jax.numpy as jnp
from jax.experimental import pallas as pl


def kernel(x, edge_index, W1, b1, W2, b2):
    raise NotImplementedError("write your pallas kernel here")


# speedup vs baseline: 37.7162x; 37.7162x over previous
import functools

import jax
import jax.numpy as jnp
import numpy as np
from jax import lax
from jax.experimental import pallas as pl
from jax.experimental.pallas import tpu as pltpu
from jax.experimental.pallas import tpu_sc as plsc

NN = 10000
EE = 320000
FF = 128
HH = 128
CC = 16

NC = 2
NS = 16
NW = NC * NS
CH = 125
NCHUNK = 80
NCH2 = 160
RING = 8
RINGH = 4
NP = 10240
RPT = NP // NS
ZR = 128


def _threefry2x32_np(k1, k2, x0, x1):
    rotations = [np.uint32(r) for r in (13, 15, 26, 6, 17, 29, 16, 24)]
    ks0, ks1 = np.uint32(k1), np.uint32(k2)
    ks2 = np.uint32(np.uint32(0x1BD11BDA) ^ ks0 ^ ks1)
    ks = [ks0, ks1, ks2]
    x0 = x0 + ks0
    x1 = x1 + ks1

    def rotl(x, d):
        return (x << d) | (x >> np.uint32(32 - int(d)))

    for i in range(5):
        rots = rotations[:4] if i % 2 == 0 else rotations[4:]
        for r in rots:
            x0 = x0 + x1
            x1 = rotl(x1, r)
            x1 = x1 ^ x0
        x0 = x0 + ks[(i + 1) % 3]
        x1 = x1 + ks[(i + 2) % 3] + np.uint32(i + 1)
    return x0, x1


def _dropout_scale(shape, seed, p):
    n = int(np.prod(shape))
    idx = np.arange(n, dtype=np.uint64)
    c1 = (idx >> np.uint64(32)).astype(np.uint32)
    c2 = (idx & np.uint64(0xFFFFFFFF)).astype(np.uint32)
    with np.errstate(over="ignore"):
        b1, b2 = _threefry2x32_np(np.uint32(seed >> 32),
                                  np.uint32(seed & 0xFFFFFFFF), c1, c2)
    bits = b1 ^ b2
    fbits = ((bits >> np.uint32(9)) | np.uint32(0x3F800000)).view(np.float32)
    u = fbits - np.float32(1.0)
    return ((u < np.float32(p)).astype(np.float32) / np.float32(p)
            ).reshape(shape)


_DROP_SCALE = _dropout_scale((NN, HH), 42, 0.5)


def _make_sc_scatter16(gather):
    d = 16
    mesh = plsc.VectorSubcoreMesh(core_axis_name="c", subcore_axis_name="s")
    scratch = [
        pltpu.VMEM((NCHUNK, CH), jnp.int32),
        pltpu.VMEM((RING, CH, d), jnp.float32),
        pltpu.VMEM((ZR, d), jnp.float32),
        pltpu.VMEM_SHARED((NP, d), jnp.float32),
    ] + [pltpu.SemaphoreType.DMA] * RING
    if gather:
        scratch.insert(0, pltpu.VMEM((NCHUNK, CH), jnp.int32))

    @functools.partial(
        pl.kernel,
        out_type=jax.ShapeDtypeStruct((NC, NP, d), jnp.float32),
        mesh=mesh,
        scratch_types=scratch,
        compiler_params=pltpu.CompilerParams(use_tc_tiling_on_sc=False),
    )
    def scat(*args):
        if gather:
            (table_hbm, src_hbm, dst_hbm, out_hbm,
             src_v, dst_v, ring_v, zbuf_v, acc_sh, *sems) = args
        else:
            (dst_hbm, out_hbm, dst_v, ring_v, zbuf_v, acc_sh,
             *sems) = args
        rows_v = ring_v.at[0]
        c = lax.axis_index("c")
        s = lax.axis_index("s")
        wid = s * NC + c

        if gather:
            pltpu.sync_copy(src_hbm.at[wid], src_v)
        pltpu.sync_copy(dst_hbm.at[wid], dst_v)

        zeros16 = jnp.zeros((16,), jnp.float32)
        ones16 = jnp.ones((16,), jnp.float32)

        def zrow(i, carry):
            zbuf_v[i, pl.ds(0, 16)] = zeros16
            return carry

        lax.fori_loop(0, ZR, zrow, 0)
        if not gather:
            def orow(i, carry):
                rows_v[i, pl.ds(0, 16)] = ones16
                return carry

            lax.fori_loop(0, CH, orow, 0)
        for k in range(RPT // ZR):
            pltpu.sync_copy(zbuf_v, acc_sh.at[pl.ds(s * RPT + k * ZR, ZR)])
        plsc.subcore_barrier()

        if gather:
            dummy = table_hbm.at[pl.ds(0, CH)]
            g4 = NCHUNK // RING
            bufs = [ring_v.at[b] for b in range(RING)]
            for b in range(RING):
                pltpu.async_copy(table_hbm.at[src_v.at[b]], bufs[b],
                                 sems[b])

            def body(g, carry):
                for b in range(RING):
                    pltpu.make_async_copy(dummy, bufs[b], sems[b]).wait()
                    pltpu.async_copy(bufs[b],
                                     acc_sh.at[dst_v.at[RING * g + b]],
                                     sems[b], add=True)

                @pl.when(g < g4 - 1)
                def _():
                    for b in range(RING):
                        pltpu.make_async_copy(dummy, bufs[b],
                                              sems[b]).wait()
                        pltpu.async_copy(
                            table_hbm.at[src_v.at[RING * g + b + RING]],
                            bufs[b], sems[b])
                return carry

            lax.fori_loop(0, g4, body, 0)
            for b in range(RING):
                pltpu.make_async_copy(dummy, bufs[b], sems[b]).wait()
        else:
            def body(ci, carry):
                pltpu.sync_copy(rows_v, acc_sh.at[dst_v.at[ci]], add=True)
                return carry

            lax.fori_loop(0, NCHUNK, body, 0)
        plsc.subcore_barrier()
        pltpu.sync_copy(acc_sh.at[pl.ds(s * RPT, RPT)],
                        out_hbm.at[c, pl.ds(s * RPT, RPT)])

    return scat


def _make_sc_scatter_h():
    dh = HH // 2
    mesh = plsc.VectorSubcoreMesh(core_axis_name="c", subcore_axis_name="s")

    @functools.partial(
        pl.kernel,
        out_type=jax.ShapeDtypeStruct((NC, NP, HH // 2), jnp.float32),
        mesh=mesh,
        scratch_types=[
            pltpu.VMEM((NCH2, CH), jnp.int32),
            pltpu.VMEM((NCH2, CH), jnp.int32),
            pltpu.VMEM((RINGH, CH, dh), jnp.float32),
            pltpu.VMEM((ZR, dh), jnp.float32),
            pltpu.VMEM_SHARED((NP, dh), jnp.float32),
        ] + [pltpu.SemaphoreType.DMA] * RINGH,
        compiler_params=pltpu.CompilerParams(use_tc_tiling_on_sc=False),
    )
    def scat(table_hbm, src_hbm, dst_hbm, out_hbm,
             src_v, dst_v, ring_v, zbuf_v, acc_sh, *sems):
        c = lax.axis_index("c")
        s = lax.axis_index("s")

        for k in range(2):
            half = pl.ds(k * NCHUNK, NCHUNK)
            pltpu.sync_copy(src_hbm.at[2 * s + k], src_v.at[half])
            pltpu.sync_copy(dst_hbm.at[2 * s + k], dst_v.at[half])

        zeros16 = jnp.zeros((16,), jnp.float32)

        def zrow(i, carry):
            for j in range(dh // 16):
                zbuf_v[i, pl.ds(j * 16, 16)] = zeros16
            return carry

        lax.fori_loop(0, ZR, zrow, 0)
        for k in range(RPT // ZR):
            pltpu.sync_copy(zbuf_v, acc_sh.at[pl.ds(s * RPT + k * ZR, ZR)])
        plsc.subcore_barrier()

        tb = table_hbm.at[c]
        dummy = tb.at[pl.ds(0, CH)]
        g4 = NCH2 // RINGH
        bufs = [ring_v.at[b] for b in range(RINGH)]
        for b in range(RINGH):
            pltpu.async_copy(tb.at[src_v.at[b]], bufs[b], sems[b])

        def body(g, carry):
            for b in range(RINGH):
                pltpu.make_async_copy(dummy, bufs[b], sems[b]).wait()
                pltpu.async_copy(bufs[b],
                                 acc_sh.at[dst_v.at[RINGH * g + b]],
                                 sems[b], add=True)

            @pl.when(g < g4 - 1)
            def _():
                for b in range(RINGH):
                    pltpu.make_async_copy(dummy, bufs[b], sems[b]).wait()
                    pltpu.async_copy(tb.at[src_v.at[RINGH * g + b + RINGH]],
                                     bufs[b], sems[b])
            return carry

        lax.fori_loop(0, g4, body, 0)
        for b in range(RINGH):
            pltpu.make_async_copy(dummy, bufs[b], sems[b]).wait()
        plsc.subcore_barrier()
        pltpu.sync_copy(acc_sh.at[pl.ds(s * RPT, RPT)],
                        out_hbm.at[c, pl.ds(s * RPT, RPT)])

    return scat


_sc_scatter_hist = _make_sc_scatter16(gather=False)
_sc_scatter_h = _make_sc_scatter_h()
_sc_scatter_c = _make_sc_scatter16(gather=True)


def _tc_a(x_ref, w1_ref, hist_ref, g1_ref, dinv_ref):
    deg = hist_ref[0, :NN, 0:1] + hist_ref[1, :NN, 0:1] + 1.0
    dinv = lax.rsqrt(deg)
    h = jnp.dot(x_ref[...], w1_ref[...], preferred_element_type=jnp.float32)
    g = h * dinv
    g1_ref[0] = g[:, :HH // 2]
    g1_ref[1] = g[:, HH // 2:]
    dinv_ref[...] = jnp.broadcast_to(dinv, (NN, 16))


def _tc_b(a1_ref, g1_ref, dinv_ref, b1_ref, drop_ref, w2_ref, g2_ref):
    dinv = dinv_ref[:, 0:1]
    g1 = jnp.concatenate([g1_ref[0], g1_ref[1]], axis=1)
    a1 = jnp.concatenate([a1_ref[0, :NN], a1_ref[1, :NN]], axis=1)
    out1 = (a1 + g1) * dinv + b1_ref[...]
    h = jnp.maximum(out1, 0.0) * drop_ref[...]
    g2_ref[...] = jnp.dot(h, w2_ref[...],
                          preferred_element_type=jnp.float32) * dinv


def _tc_c(acc2_ref, g2_ref, dinv_ref, b2_ref, out_ref):
    dinv = dinv_ref[:, 0:1]
    out_ref[...] = (acc2_ref[0, :NN] + acc2_ref[1, :NN] + g2_ref[...]) \
        * dinv + b2_ref[...]


def kernel(x, edge_index, W1, b1, W2, b2):
    src = edge_index[0].reshape(NW, NCHUNK, CH)
    dst = edge_index[1].reshape(NW, NCHUNK, CH)

    hist = _sc_scatter_hist(dst)

    g1_halves, dinv = pl.pallas_call(
        _tc_a,
        out_shape=[
            jax.ShapeDtypeStruct((NC, NN, HH // 2), jnp.float32),
            jax.ShapeDtypeStruct((NN, 16), jnp.float32),
        ],
    )(x, W1, hist)

    acc1 = _sc_scatter_h(g1_halves, src, dst)

    drop = jnp.asarray(_DROP_SCALE)

    g2 = pl.pallas_call(
        _tc_b,
        out_shape=jax.ShapeDtypeStruct((NN, CC), jnp.float32),
    )(acc1, g1_halves, dinv, b1.reshape(1, HH), drop, W2)

    acc2 = _sc_scatter_c(g2, src, dst)

    out = pl.pallas_call(
        _tc_c,
        out_shape=jax.ShapeDtypeStruct((NN, CC), jnp.float32),
    )(acc2, g2, dinv, b2.reshape(1, CC))

    return out
